# baseline (speedup 1.0000x reference)
import functools

import jax
import jax.numpy as jnp
from jax import lax
from jax.experimental import pallas as pl
from jax.experimental.pallas import tpu as pltpu
from jax.experimental.pallas import tpu_sc as plsc

VOCAB = 1000000
DIM = 64
B = 16384
L = 50

_N_FLAT = B * L

_info = plsc.get_sparse_core_info()
_NC, _NS = _info.num_cores, _info.num_subcores
_NW = _NC * _NS
_BLK = 128
_NBLK_TOTAL = B // _BLK
_BLK_PER_W = _NBLK_TOTAL // _NW
_UNITS = _BLK_PER_W * L
_IDX_PER_W = _BLK_PER_W * _BLK


def _make_gather():
    mesh = plsc.VectorSubcoreMesh(core_axis_name="c", subcore_axis_name="s")

    @functools.partial(
        pl.kernel,
        mesh=mesh,
        out_type=jax.ShapeDtypeStruct((L, 8, _NBLK_TOTAL, 1024), jnp.float32),
        scratch_types=[
            pltpu.VMEM((L * _IDX_PER_W,), jnp.int32),
            pltpu.VMEM((_BLK, DIM), jnp.float32),
            pltpu.VMEM((_BLK, DIM), jnp.float32),
            pltpu.VMEM((8, 1024), jnp.float32),
            pltpu.VMEM((8, 1024), jnp.float32),
            pltpu.SemaphoreType.DMA,
            pltpu.SemaphoreType.DMA,
            pltpu.SemaphoreType.DMA,
            pltpu.SemaphoreType.DMA,
            pltpu.SemaphoreType.DMA,
        ],
        compiler_params=pltpu.CompilerParams(
            use_tc_tiling_on_sc=False, needs_layout_passes=False
        ),
    )
    def gather_kernel(idx_hbm, table_hbm, out_hbm, idx_v, rows_v0, rows_v1,
                      tr_v0, tr_v1, g_sem0, g_sem1, w_sem0, w_sem1, i_sem):
        rows_v = (rows_v0, rows_v1)
        tr_v = (tr_v0, tr_v1)
        g_sem = (g_sem0, g_sem1)
        w_sem = (w_sem0, w_sem1)
        wid = lax.axis_index("s") * _NC + lax.axis_index("c")
        blk0 = wid * _BLK_PER_W

        for l in range(L):
            pltpu.async_copy(
                idx_hbm.at[pl.ds(l * B + blk0 * _BLK, _IDX_PER_W)],
                idx_v.at[pl.ds(l * _IDX_PER_W, _IDX_PER_W)],
                i_sem,
            )
        for l in range(L):
            pltpu.make_async_copy(
                idx_hbm.at[pl.ds(l * B + blk0 * _BLK, _IDX_PER_W)],
                idx_v.at[pl.ds(l * _IDX_PER_W, _IDX_PER_W)],
                i_sem,
            ).wait()

        def idx_slice(u):
            bl_ = u // L
            l_ = u - bl_ * L
            return idx_v.at[pl.ds(l_ * _IDX_PER_W + bl_ * _BLK, _BLK)]

        def start_gather(u, b):
            pltpu.async_copy(table_hbm.at[idx_slice(u)], rows_v[b], g_sem[b])

        def wait_gather(u, b):
            pltpu.make_async_copy(
                table_hbm.at[idx_slice(u)], rows_v[b], g_sem[b]).wait()

        def start_wb(u, b):
            bl_ = u // L
            l_ = u - bl_ * L
            pltpu.async_copy(tr_v[b], out_hbm.at[l_, :, blk0 + bl_], w_sem[b])

        def wait_wb(u, b):
            bl_ = u // L
            l_ = u - bl_ * L
            pltpu.make_async_copy(
                tr_v[b], out_hbm.at[l_, :, blk0 + bl_], w_sem[b]).wait()

        def transpose(b):
            i16 = lax.iota(jnp.int32, 16)
            for bl in range(_BLK):
                for c0 in range(0, DIM, 16):
                    v = rows_v[b][bl, pl.ds(c0, 16)]
                    c = c0 + i16
                    plsc.store_scatter(
                        tr_v[b], [c >> 3, ((c & 7) << 7) + bl], v)

        start_gather(0, 0)

        def body(p, carry):
            for b in range(2):
                u = 2 * p + b

                @pl.when(u < _UNITS - 1)
                def _():
                    start_gather(u + 1, 1 - b)

                wait_gather(u, b)

                @pl.when(u >= 2)
                def _():
                    wait_wb(u - 2, b)

                transpose(b)
                start_wb(u, b)
            return carry

        lax.fori_loop(0, _UNITS // 2, body, 0)

        wait_wb(_UNITS - 2, 0)
        wait_wb(_UNITS - 1, 1)

    return gather_kernel


_gather = _make_gather()


def kernel(x, table):
    idx_t = x.T.reshape(_N_FLAT).astype(jnp.int32)
    out5 = _gather(idx_t, table)
    out = (
        out5.reshape(L, 8, _NBLK_TOTAL, 8, _BLK)
        .transpose(2, 4, 0, 1, 3)
        .reshape(B, L, DIM)
    )
    return out


# speedup vs baseline: 1.0002x; 1.0002x over previous
import functools

import jax
import jax.numpy as jnp
from jax import lax
from jax.experimental import pallas as pl
from jax.experimental.pallas import tpu as pltpu
from jax.experimental.pallas import tpu_sc as plsc

VOCAB = 1000000
DIM = 64
B = 16384
L = 50

_N_FLAT = B * L

_info = plsc.get_sparse_core_info()
_NC, _NS = _info.num_cores, _info.num_subcores
_NW = _NC * _NS
_BLK = 128
_NBLK_TOTAL = B // _BLK
_BLK_PER_W = _NBLK_TOTAL // _NW
_UNITS = _BLK_PER_W * L
_IDX_PER_W = _BLK_PER_W * _BLK


def _make_gather():
    mesh = plsc.VectorSubcoreMesh(core_axis_name="c", subcore_axis_name="s")

    @functools.partial(
        pl.kernel,
        mesh=mesh,
        out_type=jax.ShapeDtypeStruct((L, 8, _NBLK_TOTAL, 1024), jnp.float32),
        scratch_types=[
            pltpu.VMEM((L * _IDX_PER_W,), jnp.int32),
            pltpu.VMEM((_BLK, DIM), jnp.float32),
            pltpu.VMEM((_BLK, DIM), jnp.float32),
            pltpu.VMEM((8, 1024), jnp.float32),
            pltpu.VMEM((8, 1024), jnp.float32),
            pltpu.SemaphoreType.DMA,
            pltpu.SemaphoreType.DMA,
            pltpu.SemaphoreType.DMA,
            pltpu.SemaphoreType.DMA,
            pltpu.SemaphoreType.DMA,
        ],
        compiler_params=pltpu.CompilerParams(
            use_tc_tiling_on_sc=False, needs_layout_passes=False
        ),
    )
    def gather_kernel(idx_hbm, table_hbm, out_hbm, idx_v, rows_v0, rows_v1,
                      tr_v0, tr_v1, g_sem0, g_sem1, w_sem0, w_sem1, i_sem):
        rows_v = (rows_v0, rows_v1)
        tr_v = (tr_v0, tr_v1)
        g_sem = (g_sem0, g_sem1)
        w_sem = (w_sem0, w_sem1)
        wid = lax.axis_index("s") * _NC + lax.axis_index("c")
        blk0 = wid * _BLK_PER_W

        for l in range(L):
            pltpu.async_copy(
                idx_hbm.at[pl.ds(l * B + blk0 * _BLK, _IDX_PER_W)],
                idx_v.at[pl.ds(l * _IDX_PER_W, _IDX_PER_W)],
                i_sem,
            )
        for l in range(L):
            pltpu.make_async_copy(
                idx_hbm.at[pl.ds(l * B + blk0 * _BLK, _IDX_PER_W)],
                idx_v.at[pl.ds(l * _IDX_PER_W, _IDX_PER_W)],
                i_sem,
            ).wait()

        def idx_slice(u):
            bl_ = u // L
            l_ = u - bl_ * L
            return idx_v.at[pl.ds(l_ * _IDX_PER_W + bl_ * _BLK, _BLK)]

        def start_gather(u, b):
            pltpu.async_copy(table_hbm.at[idx_slice(u)], rows_v[b], g_sem[b])

        def wait_gather(u, b):
            pltpu.make_async_copy(
                table_hbm.at[idx_slice(u)], rows_v[b], g_sem[b]).wait()

        def start_wb(u, b):
            bl_ = u // L
            l_ = u - bl_ * L
            pltpu.async_copy(tr_v[b], out_hbm.at[l_, :, blk0 + bl_], w_sem[b])

        def wait_wb(u, b):
            bl_ = u // L
            l_ = u - bl_ * L
            pltpu.make_async_copy(
                tr_v[b], out_hbm.at[l_, :, blk0 + bl_], w_sem[b]).wait()

        def transpose(b):
            i16 = lax.iota(jnp.int32, 16)
            c0s = tuple(range(0, DIM, 16))
            rowvecs = [(c0 + i16) >> 3 for c0 in c0s]
            colbases = [((c0 + i16) & 7) << 7 for c0 in c0s]
            for bl in range(_BLK):
                for ci, c0 in enumerate(c0s):
                    v = rows_v[b][bl, pl.ds(c0, 16)]
                    plsc.store_scatter(
                        tr_v[b], [rowvecs[ci], colbases[ci] + bl], v)

        start_gather(0, 0)

        def body(p, carry):
            for b in range(2):
                u = 2 * p + b

                @pl.when(u < _UNITS - 1)
                def _():
                    start_gather(u + 1, 1 - b)

                wait_gather(u, b)

                @pl.when(u >= 2)
                def _():
                    wait_wb(u - 2, b)

                transpose(b)
                start_wb(u, b)
            return carry

        lax.fori_loop(0, _UNITS // 2, body, 0)

        wait_wb(_UNITS - 2, 0)
        wait_wb(_UNITS - 1, 1)

    return gather_kernel


_gather = _make_gather()


def kernel(x, table):
    idx_t = x.T.reshape(_N_FLAT).astype(jnp.int32)
    out5 = _gather(idx_t, table)
    out = (
        out5.reshape(L, 8, _NBLK_TOTAL, 8, _BLK)
        .transpose(2, 4, 0, 1, 3)
        .reshape(B, L, DIM)
    )
    return out


# speedup vs baseline: 1.2788x; 1.2786x over previous
import functools

import jax
import jax.numpy as jnp
from jax import lax
from jax.experimental import pallas as pl
from jax.experimental.pallas import tpu as pltpu
from jax.experimental.pallas import tpu_sc as plsc

VOCAB = 1000000
DIM = 64
B = 16384
L = 50

_N_FLAT = B * L

_info = plsc.get_sparse_core_info()
_NC, _NS = _info.num_cores, _info.num_subcores
_NW = _NC * _NS
_PER_W = _N_FLAT // _NW
_NBUF = 4
_CHUNK = 400
_N_CHUNKS = _PER_W // _CHUNK
_N_GROUPS = _N_CHUNKS // _NBUF


def _make_gather():
    mesh = plsc.VectorSubcoreMesh(core_axis_name="c", subcore_axis_name="s")

    @functools.partial(
        pl.kernel,
        mesh=mesh,
        out_type=jax.ShapeDtypeStruct((_N_FLAT, DIM), jnp.float32),
        scratch_types=[
            pltpu.VMEM((_PER_W,), jnp.int32),
            pltpu.VMEM((_NBUF, _CHUNK, DIM), jnp.float32),
            pltpu.SemaphoreType.DMA((_NBUF,)),
            pltpu.SemaphoreType.DMA((_NBUF,)),
        ],
        compiler_params=pltpu.CompilerParams(use_tc_tiling_on_sc=False),
    )
    def gather_kernel(idx_hbm, table_hbm, out_hbm, idx_v, rows_v, g_sem, s_sem):
        wid = lax.axis_index("s") * _NC + lax.axis_index("c")
        w_base = wid * _PER_W

        pltpu.sync_copy(idx_hbm.at[pl.ds(w_base, _PER_W)], idx_v)

        def start_gather(j, b):
            pltpu.async_copy(
                table_hbm.at[idx_v.at[pl.ds(j * _CHUNK, _CHUNK)]],
                rows_v.at[b],
                g_sem.at[b],
            )

        def drain_and_scatter(j, b):
            pltpu.make_async_copy(
                table_hbm.at[idx_v.at[pl.ds(j * _CHUNK, _CHUNK)]],
                rows_v.at[b],
                g_sem.at[b],
            ).wait()
            pltpu.async_copy(
                rows_v.at[b],
                out_hbm.at[pl.ds(w_base + j * _CHUNK, _CHUNK)],
                s_sem.at[b],
            )

        def wait_scatter(j, b):
            pltpu.make_async_copy(
                rows_v.at[b],
                out_hbm.at[pl.ds(w_base + j * _CHUNK, _CHUNK)],
                s_sem.at[b],
            ).wait()

        for b in range(_NBUF):
            start_gather(b, b)
        for b in range(_NBUF):
            drain_and_scatter(b, b)

        def body(p, carry):
            for b in range(_NBUF):
                j = p * _NBUF + b
                wait_scatter(j - _NBUF, b)
                start_gather(j, b)
            for b in range(_NBUF):
                j = p * _NBUF + b
                drain_and_scatter(j, b)
            return carry

        lax.fori_loop(1, _N_GROUPS, body, 0)

        for b in range(_NBUF):
            wait_scatter((_N_GROUPS - 1) * _NBUF + b, b)

    return gather_kernel


_gather = _make_gather()


def kernel(x, table):
    flat_idx = x.reshape(_N_FLAT).astype(jnp.int32)
    out = _gather(flat_idx, table)
    return out.reshape(B, L, DIM)
